# baseline (speedup 1.0000x reference)
import jax
import jax.numpy as jnp
from jax.experimental import pallas as pl
from jax.experimental.pallas import tpu as pltpu

N = 224
HID = 128
NL = 4
STRIDE = 15
BATCH = 1024
BT = 128


def _normalize(h):
    hb = h.astype(jnp.bfloat16)
    ones_m = jnp.full((HID, HID), 1.0 / HID, dtype=jnp.bfloat16)
    m = jnp.dot(hb, ones_m, preferred_element_type=jnp.float32)
    q = jnp.dot(hb * hb, ones_m, preferred_element_type=jnp.float32)
    d = h - m
    v = q - m * m
    return d * jax.lax.rsqrt(v + 1e-5)


def _gnn_kernel(xt_ref, p0_ref, bias0_ref, WT_ref, WB_ref, bias_ref,
                w1_ref, b1_ref, w2_ref, b2_ref, out_ref):
    R = N * BT
    xt = xt_ref[...]

    zx = jnp.zeros((1, BT), dtype=jnp.float32)
    zx15 = jnp.zeros((STRIDE, BT), dtype=jnp.float32)
    aggx = (jnp.concatenate([xt[1:], zx], axis=0)
            + jnp.concatenate([zx, xt[:-1]], axis=0)
            + jnp.concatenate([xt[STRIDE:], zx15], axis=0)
            + jnp.concatenate([zx15, xt[:-STRIDE]], axis=0))
    nvec = jax.lax.broadcasted_iota(jnp.int32, (N, 1), 0)
    degn = ((nvec >= 1).astype(jnp.float32)
            + (nvec < N - 1).astype(jnp.float32)
            + (nvec >= STRIDE).astype(jnp.float32)
            + (nvec < N - STRIDE).astype(jnp.float32))
    aggx = aggx * (1.0 / degn)
    xc = jnp.concatenate([xt.reshape(N, 1, BT), aggx.reshape(N, 1, BT)],
                         axis=1)
    z = jax.lax.dot_general(
        xc, p0_ref[...],
        dimension_numbers=(((1,), (0,)), ((), ())),
        preferred_element_type=jnp.float32).reshape(R, HID)
    u = _normalize(jnp.maximum(z + bias0_ref[0][None, :], 0.0))

    n_idx = jax.lax.broadcasted_iota(jnp.int32, (R, 1), 0) // BT
    deg = ((n_idx >= 1).astype(jnp.float32)
           + (n_idx < N - 1).astype(jnp.float32)
           + (n_idx >= STRIDE).astype(jnp.float32)
           + (n_idx < N - STRIDE).astype(jnp.float32))
    inv_deg = 1.0 / deg

    zpad = jnp.zeros((STRIDE * BT, HID), dtype=jnp.float32)

    for l in range(NL - 1):
        P = jnp.concatenate([zpad, u, zpad], axis=0)
        agg = (P[(STRIDE + 1) * BT:(STRIDE + 1) * BT + R]
               + P[(STRIDE - 1) * BT:(STRIDE - 1) * BT + R]
               + P[2 * STRIDE * BT:2 * STRIDE * BT + R]
               + P[:R])
        agg = agg * inv_deg
        z = (jnp.dot(u, WT_ref[l], preferred_element_type=jnp.float32)
             + jnp.dot(agg, WB_ref[l], preferred_element_type=jnp.float32)
             + bias_ref[l][None, :])
        u = _normalize(jnp.maximum(z, 0.0))

    graph = jnp.mean(u.reshape(N, BT, HID), axis=0)
    hidden = jnp.maximum(
        jnp.dot(graph, w1_ref[...], preferred_element_type=jnp.float32)
        + b1_ref[0][None, :], 0.0)
    out_ref[...] = (jnp.dot(hidden, w2_ref[...],
                            preferred_element_type=jnp.float32)
                    + b2_ref[0][None, :])


def kernel(x, ne_w, ne_b, gnn_W, gnn_b, ln_g, ln_b,
           mlp_w1, mlp_b1, mlp_w2, mlp_b2):
    Wt = gnn_W[:, :HID, :]
    Wb = gnn_W[:, HID:, :]
    w = ne_w[0]
    p0 = jnp.stack([w @ Wt[0], w @ Wb[0]])
    bias0 = (gnn_b[0] + ne_b @ (Wt[0] + Wb[0])).reshape(1, HID)
    WT = ln_g[:NL - 1, :, None] * Wt[1:]
    WB = ln_g[:NL - 1, :, None] * Wb[1:]
    bias = gnn_b[1:] + jnp.einsum('lk,lkh->lh', ln_b[:NL - 1],
                                  Wt[1:] + Wb[1:])
    w1 = ln_g[NL - 1][:, None] * mlp_w1
    b1 = (mlp_b1 + ln_b[NL - 1] @ mlp_w1).reshape(1, HID)

    grid = (BATCH // BT,)
    rep = lambda shape: pl.BlockSpec(shape, lambda i: (0,) * len(shape))
    return pl.pallas_call(
        _gnn_kernel,
        grid=grid,
        in_specs=[
            pl.BlockSpec((N, BT), lambda i: (0, i)),
            rep((2, HID)),
            rep((1, HID)),
            rep((NL - 1, HID, HID)),
            rep((NL - 1, HID, HID)),
            rep((NL - 1, HID)),
            rep((HID, HID)),
            rep((1, HID)),
            rep((HID, 2)),
            rep((1, 2)),
        ],
        out_specs=pl.BlockSpec((BT, 2), lambda i: (i, 0)),
        out_shape=jax.ShapeDtypeStruct((BATCH, 2), jnp.float32),
        compiler_params=pltpu.CompilerParams(
            dimension_semantics=("parallel",),
        ),
    )(x.T, p0, bias0, WT, WB, bias, w1, b1, mlp_w2, mlp_b2.reshape(1, 2))


# speedup vs baseline: 1.1319x; 1.1319x over previous
import jax
import jax.numpy as jnp
from jax.experimental import pallas as pl
from jax.experimental.pallas import tpu as pltpu

N = 224
HID = 128
NL = 4
STRIDE = 15
BATCH = 1024
BT = 256


def _normalize(h):
    m = jnp.mean(h, axis=1, keepdims=True)
    d = h - m
    v = jnp.mean(d * d, axis=1, keepdims=True)
    return d * jax.lax.rsqrt(v + 1e-5)


def _gnn_kernel(xt_ref, p0_ref, bias0_ref, WT_ref, WB_ref, bias_ref,
                w1_ref, b1_ref, w2_ref, b2_ref, out_ref):
    R = N * BT
    xt = xt_ref[...]

    zx = jnp.zeros((1, BT), dtype=jnp.float32)
    zx15 = jnp.zeros((STRIDE, BT), dtype=jnp.float32)
    aggx = (jnp.concatenate([xt[1:], zx], axis=0)
            + jnp.concatenate([zx, xt[:-1]], axis=0)
            + jnp.concatenate([xt[STRIDE:], zx15], axis=0)
            + jnp.concatenate([zx15, xt[:-STRIDE]], axis=0))
    nvec = jax.lax.broadcasted_iota(jnp.int32, (N, 1), 0)
    degn = ((nvec >= 1).astype(jnp.float32)
            + (nvec < N - 1).astype(jnp.float32)
            + (nvec >= STRIDE).astype(jnp.float32)
            + (nvec < N - STRIDE).astype(jnp.float32))
    aggx = aggx * (1.0 / degn)
    xc = jnp.concatenate([xt.reshape(N, 1, BT), aggx.reshape(N, 1, BT)],
                         axis=1)
    z = jax.lax.dot_general(
        xc, p0_ref[...],
        dimension_numbers=(((1,), (0,)), ((), ())),
        preferred_element_type=jnp.float32).reshape(R, HID)
    u = _normalize(jnp.maximum(z + bias0_ref[0][None, :], 0.0))

    n_idx = jax.lax.broadcasted_iota(jnp.int32, (R, 1), 0) // BT
    deg = ((n_idx >= 1).astype(jnp.float32)
           + (n_idx < N - 1).astype(jnp.float32)
           + (n_idx >= STRIDE).astype(jnp.float32)
           + (n_idx < N - STRIDE).astype(jnp.float32))
    inv_deg = 1.0 / deg

    zpad = jnp.zeros((STRIDE * BT, HID), dtype=jnp.float32)

    for l in range(NL - 1):
        P = jnp.concatenate([zpad, u, zpad], axis=0)
        agg = (P[(STRIDE + 1) * BT:(STRIDE + 1) * BT + R]
               + P[(STRIDE - 1) * BT:(STRIDE - 1) * BT + R]
               + P[2 * STRIDE * BT:2 * STRIDE * BT + R]
               + P[:R])
        agg = agg * inv_deg
        z = (jnp.dot(u, WT_ref[l], preferred_element_type=jnp.float32)
             + jnp.dot(agg, WB_ref[l], preferred_element_type=jnp.float32)
             + bias_ref[l][None, :])
        u = _normalize(jnp.maximum(z, 0.0))

    graph = jnp.mean(u.reshape(N, BT, HID), axis=0)
    hidden = jnp.maximum(
        jnp.dot(graph, w1_ref[...], preferred_element_type=jnp.float32)
        + b1_ref[0][None, :], 0.0)
    out_ref[...] = (jnp.dot(hidden, w2_ref[...],
                            preferred_element_type=jnp.float32)
                    + b2_ref[0][None, :])


def kernel(x, ne_w, ne_b, gnn_W, gnn_b, ln_g, ln_b,
           mlp_w1, mlp_b1, mlp_w2, mlp_b2):
    Wt = gnn_W[:, :HID, :]
    Wb = gnn_W[:, HID:, :]
    w = ne_w[0]
    p0 = jnp.stack([w @ Wt[0], w @ Wb[0]])
    bias0 = (gnn_b[0] + ne_b @ (Wt[0] + Wb[0])).reshape(1, HID)
    WT = ln_g[:NL - 1, :, None] * Wt[1:]
    WB = ln_g[:NL - 1, :, None] * Wb[1:]
    bias = gnn_b[1:] + jnp.einsum('lk,lkh->lh', ln_b[:NL - 1],
                                  Wt[1:] + Wb[1:])
    w1 = ln_g[NL - 1][:, None] * mlp_w1
    b1 = (mlp_b1 + ln_b[NL - 1] @ mlp_w1).reshape(1, HID)

    grid = (BATCH // BT,)
    rep = lambda shape: pl.BlockSpec(shape, lambda i: (0,) * len(shape))
    return pl.pallas_call(
        _gnn_kernel,
        grid=grid,
        in_specs=[
            pl.BlockSpec((N, BT), lambda i: (0, i)),
            rep((2, HID)),
            rep((1, HID)),
            rep((NL - 1, HID, HID)),
            rep((NL - 1, HID, HID)),
            rep((NL - 1, HID)),
            rep((HID, HID)),
            rep((1, HID)),
            rep((HID, 2)),
            rep((1, 2)),
        ],
        out_specs=pl.BlockSpec((BT, 2), lambda i: (i, 0)),
        out_shape=jax.ShapeDtypeStruct((BATCH, 2), jnp.float32),
        compiler_params=pltpu.CompilerParams(
            dimension_semantics=("parallel",),
            vmem_limit_bytes=128 * 1024 * 1024,
        ),
    )(x.T, p0, bias0, WT, WB, bias, w1, b1, mlp_w2, mlp_b2.reshape(1, 2))
